# baseline (speedup 1.0000x reference)
import jax
import jax.numpy as jnp
from jax import lax
from jax.experimental import pallas as pl
from jax.experimental.pallas import tpu as pltpu
from jax.experimental.pallas import tpu_sc as plsc

Z = 384
B = 64
K = 8448
N = 16896
EXT_ROWS = 20
NBLK = 26
DBL = 2 * Z
ZPAD = NBLK * DBL
CWD = ZPAD + Z
NCHUNK = Z // 16

NC = 1
NS = 16
ROWS_PER_W = B // (NC * NS)


def _par2(x):
    return (x.astype(jnp.int32) & 1).astype(jnp.float32)


def _sc_body(bits_hbm, ab_hbm, cb_hbm, p0i_hbm, out_hbm,
             cwd_v, bits_v, ab_v, cb_v, p0i_v, m_v, mt_v, ext_v):
    wid = lax.axis_index("s") * NC + lax.axis_index("c")

    pltpu.sync_copy(ab_hbm, ab_v)
    pltpu.sync_copy(cb_hbm, cb_v)
    pltpu.sync_copy(p0i_hbm, p0i_v)

    def zero_tail(j, carry):
        cwd_v[pl.ds(ZPAD + j * 16, 16)] = jnp.zeros((16,), jnp.float32)
        return carry
    lax.fori_loop(0, NCHUNK, zero_tail, 0)

    for k in range(ROWS_PER_W):
        b = wid * ROWS_PER_W + k

        pltpu.sync_copy(bits_hbm.at[pl.ds(b * K, K)], bits_v)

        def dup(c, carry):
            vs = [bits_v[pl.ds(c * Z + j * 16, 16)] for j in range(NCHUNK)]
            for j in range(NCHUNK):
                cwd_v[pl.ds(c * DBL + j * 16, 16)] = vs[j]
                cwd_v[pl.ds(c * DBL + Z + j * 16, 16)] = vs[j]
            return carry
        lax.fori_loop(0, 22, dup, 0)

        def stage1(j, carry):
            off = j * 16
            for r in range(4):
                acc = None
                for e in range(22):
                    base = ab_v[pl.ds((r * 22 + e) * 16, 16)]
                    g = plsc.load_gather(cwd_v, [base + off])
                    acc = g if acc is None else acc + g
                m_v[pl.ds(r * Z + off, 16)] = _par2(acc)
            return carry
        lax.fori_loop(0, NCHUNK, stage1, 0)

        def stage2(j, carry):
            off = j * 16
            m0 = m_v[pl.ds(0 * Z + off, 16)]
            m1 = m_v[pl.ds(1 * Z + off, 16)]
            m2 = m_v[pl.ds(2 * Z + off, 16)]
            m3 = m_v[pl.ds(3 * Z + off, 16)]
            mt_v[pl.ds(off, 16)] = _par2(m0 + m1 + m2 + m3)
            return carry
        lax.fori_loop(0, NCHUNK, stage2, 0)

        def stage2b(j, carry):
            off = j * 16
            pidx = p0i_v[pl.ds(off, 16)]
            p0 = plsc.load_gather(mt_v, [pidx])
            m1 = m_v[pl.ds(1 * Z + off, 16)]
            m2 = m_v[pl.ds(2 * Z + off, 16)]
            m3 = m_v[pl.ds(3 * Z + off, 16)]
            p1 = _par2(m1 + m2 + m3)
            p3 = _par2(m3 + p0)
            p2 = _par2(m2 + p3)
            for i, p in enumerate((p0, p1, p2, p3)):
                cwd_v[pl.ds((22 + i) * DBL + off, 16)] = p
                cwd_v[pl.ds((22 + i) * DBL + Z + off, 16)] = p
            return carry
        lax.fori_loop(0, NCHUNK, stage2b, 0)

        def stage3(j, carry):
            off = j * 16
            for r in range(EXT_ROWS):
                acc = None
                for e in range(4):
                    base = cb_v[pl.ds((r * 4 + e) * 16, 16)]
                    g = plsc.load_gather(cwd_v, [base + off])
                    acc = g if acc is None else acc + g
                ext_v[pl.ds(r * Z + off, 16)] = _par2(acc)
            return carry
        lax.fori_loop(0, NCHUNK, stage3, 0)

        ob = b * N
        pltpu.sync_copy(bits_v.at[pl.ds(2 * Z, K - 2 * Z)],
                        out_hbm.at[pl.ds(ob, K - 2 * Z)])
        for i in range(4):
            pltpu.sync_copy(cwd_v.at[pl.ds((22 + i) * DBL, Z)],
                            out_hbm.at[pl.ds(ob + K - 2 * Z + i * Z, Z)])
        pltpu.sync_copy(ext_v,
                        out_hbm.at[pl.ds(ob + K + 2 * Z, EXT_ROWS * Z)])


def kernel(inputs, A_r, A_c, A_s, C_r, C_c, C_s):
    bits = inputs.astype(jnp.float32).reshape(B * K)
    ar = jnp.asarray(A_r, jnp.int32)
    ac = jnp.asarray(A_c, jnp.int32)
    ash = jnp.asarray(A_s, jnp.int32)
    cc = jnp.asarray(C_c, jnp.int32)
    cs = jnp.asarray(C_s, jnp.int32)
    del C_r
    na = ar.shape[0]

    iota16 = jnp.arange(16, dtype=jnp.int32)
    perm = jnp.argsort(ar, stable=True)
    r_sorted = ar[perm]
    first = jnp.searchsorted(r_sorted, jnp.arange(4, dtype=jnp.int32))
    rank = jnp.arange(na, dtype=jnp.int32) - first[r_sorted]
    slots = r_sorted * 22 + rank
    a_base = (ac * DBL + ash)[perm][:, None] + iota16[None, :]
    ab = jnp.full((4 * 22, 16), ZPAD, jnp.int32).at[slots].set(a_base)
    ab = ab.at[:, :].add(0).reshape(-1)
    cb = ((cc[:4 * EXT_ROWS] * DBL + cs[:4 * EXT_ROWS])[:, None]
          + iota16[None, :]).reshape(-1)
    iota = jnp.arange(Z, dtype=jnp.int32)
    p0i = (iota + Z - 1) % Z

    mesh = plsc.VectorSubcoreMesh(core_axis_name="c", subcore_axis_name="s",
                                  num_cores=1)
    out = pl.kernel(
        _sc_body,
        out_type=jax.ShapeDtypeStruct((B * N,), jnp.float32),
        mesh=mesh,
        compiler_params=pltpu.CompilerParams(needs_layout_passes=False),
        scratch_types=[
            pltpu.VMEM((CWD,), jnp.float32),
            pltpu.VMEM((K,), jnp.float32),
            pltpu.VMEM((4 * 22 * 16,), jnp.int32),
            pltpu.VMEM((4 * EXT_ROWS * 16,), jnp.int32),
            pltpu.VMEM((Z,), jnp.int32),
            pltpu.VMEM((4 * Z,), jnp.float32),
            pltpu.VMEM((Z,), jnp.float32),
            pltpu.VMEM((EXT_ROWS * Z,), jnp.float32),
        ],
    )(bits, ab, cb, p0i)
    return out.reshape(B, N)


# speedup vs baseline: 1.8320x; 1.8320x over previous
import jax
import jax.numpy as jnp
from jax import lax
from jax.experimental import pallas as pl
from jax.experimental.pallas import tpu as pltpu
from jax.experimental.pallas import tpu_sc as plsc

Z = 384
B = 64
K = 8448
N = 16896
EXT_ROWS = 20
NBLK = 26
DBL = 2 * Z
ZPAD = NBLK * DBL
CWD = ZPAD + Z
NCHUNK = Z // 16

NC = 2
NS = 16
ROWS_PER_W = B // (NC * NS)


def _par2(x):
    return (x.astype(jnp.int32) & 1).astype(jnp.float32)


def _sc_body(bits_hbm, ab_hbm, cb_hbm, p0i_hbm, out_hbm,
             cwd_v, bits_v, ab_v, cb_v, p0i_v, m_v, mt_v, ext_v):
    wid = lax.axis_index("s") * NC + lax.axis_index("c")

    pltpu.sync_copy(ab_hbm, ab_v)
    pltpu.sync_copy(cb_hbm, cb_v)
    pltpu.sync_copy(p0i_hbm, p0i_v)

    def zero_tail(j, carry):
        cwd_v[pl.ds(ZPAD + j * 16, 16)] = jnp.zeros((16,), jnp.float32)
        return carry
    lax.fori_loop(0, NCHUNK, zero_tail, 0)

    for k in range(ROWS_PER_W):
        b = wid * ROWS_PER_W + k

        pltpu.sync_copy(bits_hbm.at[pl.ds(b * K, K)], bits_v)

        def dup(c, carry):
            vs = [bits_v[pl.ds(c * Z + j * 16, 16)] for j in range(NCHUNK)]
            for j in range(NCHUNK):
                cwd_v[pl.ds(c * DBL + j * 16, 16)] = vs[j]
                cwd_v[pl.ds(c * DBL + Z + j * 16, 16)] = vs[j]
            return carry
        lax.fori_loop(0, 22, dup, 0)

        HALF = NCHUNK // 2
        zero16 = jnp.zeros((16,), jnp.float32)
        for r in range(4):
            for h in range(2):
                def entry1(e, accs, r=r, h=h):
                    base = ab_v[pl.ds((r * 22 + e) * 16, 16)]
                    return tuple(
                        a + plsc.load_gather(cwd_v, [base + (h * HALF + j) * 16])
                        for j, a in enumerate(accs))
                accs = lax.fori_loop(0, 22, entry1, (zero16,) * HALF)
                for j, a in enumerate(accs):
                    m_v[pl.ds(r * Z + (h * HALF + j) * 16, 16)] = _par2(a)

        def stage2(j, carry):
            off = j * 16
            m0 = m_v[pl.ds(0 * Z + off, 16)]
            m1 = m_v[pl.ds(1 * Z + off, 16)]
            m2 = m_v[pl.ds(2 * Z + off, 16)]
            m3 = m_v[pl.ds(3 * Z + off, 16)]
            mt_v[pl.ds(off, 16)] = _par2(m0 + m1 + m2 + m3)
            return carry
        lax.fori_loop(0, NCHUNK, stage2, 0)

        def stage2b(j, carry):
            off = j * 16
            pidx = p0i_v[pl.ds(off, 16)]
            p0 = plsc.load_gather(mt_v, [pidx])
            m1 = m_v[pl.ds(1 * Z + off, 16)]
            m2 = m_v[pl.ds(2 * Z + off, 16)]
            m3 = m_v[pl.ds(3 * Z + off, 16)]
            p1 = _par2(m1 + m2 + m3)
            p3 = _par2(m3 + p0)
            p2 = _par2(m2 + p3)
            for i, p in enumerate((p0, p1, p2, p3)):
                cwd_v[pl.ds((22 + i) * DBL + off, 16)] = p
                cwd_v[pl.ds((22 + i) * DBL + Z + off, 16)] = p
            return carry
        lax.fori_loop(0, NCHUNK, stage2b, 0)

        def ext_row(r, carry):
            for h in range(2):
                def entry3(e, accs, h=h):
                    base = cb_v[pl.ds((r * 4 + e) * 16, 16)]
                    return tuple(
                        a + plsc.load_gather(cwd_v, [base + (h * HALF + j) * 16])
                        for j, a in enumerate(accs))
                accs = lax.fori_loop(0, 4, entry3, (zero16,) * HALF)
                for j, a in enumerate(accs):
                    ext_v[pl.ds(r * Z + (h * HALF + j) * 16, 16)] = _par2(a)
            return carry
        lax.fori_loop(0, EXT_ROWS, ext_row, 0)

        ob = b * N
        pltpu.sync_copy(bits_v.at[pl.ds(2 * Z, K - 2 * Z)],
                        out_hbm.at[pl.ds(ob, K - 2 * Z)])
        for i in range(4):
            pltpu.sync_copy(cwd_v.at[pl.ds((22 + i) * DBL, Z)],
                            out_hbm.at[pl.ds(ob + K - 2 * Z + i * Z, Z)])
        pltpu.sync_copy(ext_v,
                        out_hbm.at[pl.ds(ob + K + 2 * Z, EXT_ROWS * Z)])


def kernel(inputs, A_r, A_c, A_s, C_r, C_c, C_s):
    bits = inputs.astype(jnp.float32).reshape(B * K)
    ar = jnp.asarray(A_r, jnp.int32)
    ac = jnp.asarray(A_c, jnp.int32)
    ash = jnp.asarray(A_s, jnp.int32)
    cc = jnp.asarray(C_c, jnp.int32)
    cs = jnp.asarray(C_s, jnp.int32)
    del C_r
    na = ar.shape[0]

    iota16 = jnp.arange(16, dtype=jnp.int32)
    perm = jnp.argsort(ar, stable=True)
    r_sorted = ar[perm]
    first = jnp.searchsorted(r_sorted, jnp.arange(4, dtype=jnp.int32))
    rank = jnp.arange(na, dtype=jnp.int32) - first[r_sorted]
    slots = r_sorted * 22 + rank
    a_base = (ac * DBL + ash)[perm][:, None] + iota16[None, :]
    ab = jnp.full((4 * 22, 16), ZPAD, jnp.int32).at[slots].set(a_base)
    ab = ab.at[:, :].add(0).reshape(-1)
    cb = ((cc[:4 * EXT_ROWS] * DBL + cs[:4 * EXT_ROWS])[:, None]
          + iota16[None, :]).reshape(-1)
    iota = jnp.arange(Z, dtype=jnp.int32)
    p0i = (iota + Z - 1) % Z

    mesh = plsc.VectorSubcoreMesh(core_axis_name="c", subcore_axis_name="s")
    out = pl.kernel(
        _sc_body,
        out_type=jax.ShapeDtypeStruct((B * N,), jnp.float32),
        mesh=mesh,
        compiler_params=pltpu.CompilerParams(needs_layout_passes=False),
        scratch_types=[
            pltpu.VMEM((CWD,), jnp.float32),
            pltpu.VMEM((K,), jnp.float32),
            pltpu.VMEM((4 * 22 * 16,), jnp.int32),
            pltpu.VMEM((4 * EXT_ROWS * 16,), jnp.int32),
            pltpu.VMEM((Z,), jnp.int32),
            pltpu.VMEM((4 * Z,), jnp.float32),
            pltpu.VMEM((Z,), jnp.float32),
            pltpu.VMEM((EXT_ROWS * Z,), jnp.float32),
        ],
    )(bits, ab, cb, p0i)
    return out.reshape(B, N)
